# Initial kernel scaffold; baseline (speedup 1.0000x reference)
import jax
import jax.numpy as jnp
from jax.experimental import pallas as pl


def kernel(data, embed, W0_w, W0_b, a0, ln0_g, ln0_b, W1_w, W1_b, a1, ln1_g, ln1_b, Wo_w, Wo_b, ao, lno_g, lno_b, lino_w, lino_b, out1_w, out1_b, out2_w, out2_b):
    raise NotImplementedError("write your pallas kernel here")


# speedup vs baseline: 2.6276x; 2.6276x over previous
import math

import jax
import jax.numpy as jnp
from jax import lax
from jax.experimental import pallas as pl

B = 16
F = 511
N = 513
NP = 520
D = 128
ALPHA = 0.2
DH = math.sqrt(float(D))
EPS = 1e-6


def _leaky(x):
    return jnp.where(x >= 0, x, ALPHA * x)


def _ln(x, g, b):
    m = jnp.mean(x, axis=-1, keepdims=True)
    xc = x - m
    var = jnp.sum(xc * xc, axis=-1, keepdims=True) / (D - 1)
    return g * xc / (jnp.sqrt(var) + EPS) + b


def _elu(x):
    return jnp.where(x > 0, x, jnp.exp(x) - 1.0)


def _finalize(hp, h, rs, mcol):
    zero = rs == 0.0
    return jnp.where(zero, h, (mcol * hp) / jnp.where(zero, 1.0, rs))


def _body(imc_ref, imt_ref, omc_ref, embed_ref,
          W0w_ref, W0b_ref, a0s_ref, a0t_ref, ln0g_ref, ln0b_ref,
          W1w_ref, W1b_ref, a1s_ref, a1t_ref, ln1g_ref, ln1b_ref,
          Wow_ref, Wob_ref, aos_ref, aot_ref, lnog_ref, lnob_ref,
          linow_ref, linob_ref, out1w_ref, out1b_ref, out2w_ref, out2b_ref,
          logits_ref):
    f32 = jnp.float32

    x0 = embed_ref[...]
    h1 = jnp.dot(x0, W0w_ref[...].T, preferred_element_type=f32) + W0b_ref[...]
    s1 = lax.dot_general(h1, a0s_ref[...], (((1,), (1,)), ((), ())),
                         preferred_element_type=f32)
    t1 = lax.dot_general(a0t_ref[...], h1, (((1,), (1,)), ((), ())),
                         preferred_element_type=f32)
    E1 = jnp.exp(_leaky(s1 + t1) * (1.0 / DH))

    HM = jnp.concatenate([imc_ref[k] * h1 for k in range(B)], axis=1)
    HP = jnp.dot(E1, HM, preferred_element_type=f32)
    RS = jnp.dot(E1, imt_ref[...], preferred_element_type=f32)
    x_all = jnp.concatenate(
        [_finalize(HP[:, k * D:(k + 1) * D], h1,
                   imc_ref[k] * RS[:, k:k + 1], imc_ref[k])
         for k in range(B)], axis=0)
    x_all = _elu(_ln(x_all, ln0g_ref[...], ln0b_ref[...]))

    h_all = jnp.dot(x_all, W1w_ref[...].T, preferred_element_type=f32) \
        + W1b_ref[...]
    s_all = lax.dot_general(h_all, a1s_ref[...], (((1,), (1,)), ((), ())),
                            preferred_element_type=f32)
    hps = []
    rss = []
    for k in range(B):
        hk = h_all[k * NP:(k + 1) * NP, :]
        sk = s_all[k * NP:(k + 1) * NP, :]
        tk = lax.dot_general(a1t_ref[...], hk, (((1,), (1,)), ((), ())),
                             preferred_element_type=f32)
        E2 = jnp.exp(_leaky(sk + tk) * (1.0 / DH))
        mcol = imc_ref[k]
        hps.append(jnp.dot(E2, mcol * hk, preferred_element_type=f32))
        rss.append(mcol * jnp.dot(E2, mcol, preferred_element_type=f32))
    HP2 = jnp.concatenate(hps, axis=0)
    RS2 = jnp.concatenate(rss, axis=0)
    M_all = jnp.concatenate([imc_ref[k] for k in range(B)], axis=0)
    x_all = _finalize(HP2, h_all, RS2, M_all)
    x_all = _elu(_ln(x_all, ln1g_ref[...], ln1b_ref[...]))

    h_all = jnp.dot(x_all, Wow_ref[...].T, preferred_element_type=f32) \
        + Wob_ref[...]
    t_all = lax.dot_general(h_all, aot_ref[...], (((1,), (1,)), ((), ())),
                            preferred_element_type=f32)
    rows = []
    for k in range(B):
        hk = h_all[k * NP:(k + 1) * NP, :]
        hlast = hk[N - 1:N, :]
        s_last = jnp.sum(hlast * aos_ref[...])
        wcol = jnp.exp(_leaky(s_last + t_all[k * NP:(k + 1) * NP, :])
                       * (1.0 / DH)) * omc_ref[k]
        rs = jnp.sum(wcol)
        zero = rs == 0.0
        rs = jnp.where(zero, 1.0, rs)
        hp = lax.dot_general(wcol, hk, (((0,), (0,)), ((), ())),
                             preferred_element_type=f32)
        rows.append((hp + jnp.where(zero, hlast, 0.0)) / rs)
    z = jnp.concatenate(rows, axis=0)

    z = jnp.maximum(_ln(z, lnog_ref[...], lnob_ref[...]), 0.0)
    z = jnp.dot(z, linow_ref[...].T, preferred_element_type=f32) \
        + linob_ref[...]
    z = jnp.maximum(z, 0.0)
    z = jnp.maximum(
        jnp.dot(z, out1w_ref[...].T, preferred_element_type=f32)
        + out1b_ref[...], 0.0)
    logits_ref[...] = jnp.sum(z * out2w_ref[...], axis=1, keepdims=True) \
        + out2b_ref[0, 0]


@jax.jit
def kernel(data, embed, W0_w, W0_b, a0, ln0_g, ln0_b, W1_w, W1_b, a1,
           ln1_g, ln1_b, Wo_w, Wo_b, ao, lno_g, lno_b, lino_w, lino_b,
           out1_w, out1_b, out2_w, out2_b):
    f32 = jnp.float32
    obs = (data != 0).astype(f32)
    m = jnp.pad(obs, ((0, 0), (1, NP - F - 1)))
    any_obs = jnp.any(data != 0, axis=1, keepdims=True)
    e0 = (jnp.arange(NP) == 0).astype(f32)[None, :]
    im = jnp.where(any_obs, m, e0)
    om = m.at[:, N - 1].set(1.0)

    imc = im.reshape(B, NP, 1)
    omc = om.reshape(B, NP, 1)
    imt = im.T
    embed_p = jnp.pad(embed, ((0, NP - N), (0, 0)))

    row = lambda v: v.reshape(1, -1)
    operands = (
        imc, imt, omc, embed_p,
        W0_w, row(W0_b), a0[:, :D], a0[:, D:], row(ln0_g), row(ln0_b),
        W1_w, row(W1_b), a1[:, :D], a1[:, D:], row(ln1_g), row(ln1_b),
        Wo_w, row(Wo_b), ao[:, :D], ao[:, D:], row(lno_g), row(lno_b),
        lino_w, row(lino_b), out1_w, row(out1_b), out2_w, row(out2_b),
    )

    logits = pl.pallas_call(
        _body,
        grid=(1,),
        in_specs=[pl.BlockSpec(x.shape, lambda i, nd=x.ndim: (0,) * nd)
                  for x in operands],
        out_specs=pl.BlockSpec((B, 1), lambda i: (0, 0)),
        out_shape=jax.ShapeDtypeStruct((B, 1), f32),
    )(*operands)
    return (logits, jnp.asarray(0.0))


# speedup vs baseline: 3.0428x; 1.1580x over previous
import math

import jax
import jax.numpy as jnp
from jax import lax
from jax.experimental import pallas as pl

B = 16
F = 511
N = 513
NP = 520
D = 128
ALPHA = 0.2
DH = math.sqrt(float(D))
EPS = 1e-6


def _ln_act(x, g, b, act):
    m = jnp.mean(x, axis=-1, keepdims=True)
    xc = x - m
    var = jnp.sum(xc * xc, axis=-1, keepdims=True) / (D - 1)
    inv = 1.0 / (jnp.sqrt(var) + EPS)
    return act(g * (xc * inv) + b)


def _elu(x):
    return jnp.where(x > 0, x, jnp.exp(x) - 1.0)


def _relu(x):
    return jnp.maximum(x, 0.0)


def _body(im_ref, imstack_ref, imt_ref, omstack_ref, embed_ref,
          W0w_ref, W0b_ref, a0s_ref, a0t_ref, ln0g_ref, ln0b_ref,
          W1w_ref, W1b_ref, a1s_ref, a1t_ref, ln1g_ref, ln1b_ref,
          Wow_ref, Wob_ref, aos_ref, aot_ref, lnog_ref, lnob_ref,
          linow_ref, linob_ref, out1w_ref, out1b_ref, out2w_ref, out2b_ref,
          logits_ref):
    f32 = jnp.float32
    c = 1.0 / DH

    x0 = embed_ref[...]
    h1 = jnp.dot(x0, W0w_ref[...].T, preferred_element_type=f32) + W0b_ref[...]
    s1 = lax.dot_general(h1, a0s_ref[...], (((1,), (1,)), ((), ())),
                         preferred_element_type=f32)
    t1 = lax.dot_general(a0t_ref[...], h1, (((1,), (1,)), ((), ())),
                         preferred_element_type=f32)
    E1 = jnp.maximum(jnp.exp(s1 * c) * jnp.exp(t1 * c),
                     jnp.exp(s1 * (ALPHA * c)) * jnp.exp(t1 * (ALPHA * c)))

    HM = jnp.concatenate(
        [imstack_ref[k * NP:(k + 1) * NP, :] * h1 for k in range(B)], axis=1)
    HP = jnp.dot(E1, HM, preferred_element_type=f32)
    RS = jnp.dot(E1, imt_ref[...], preferred_element_type=f32)
    xs = []
    for k in range(B):
        mcol = imstack_ref[k * NP:(k + 1) * NP, :]
        rs = mcol * RS[:, k:k + 1]
        zero = rs == 0.0
        coef = mcol / jnp.where(zero, 1.0, rs)
        xs.append(jnp.where(zero, h1, HP[:, k * D:(k + 1) * D] * coef))
    x_all = jnp.concatenate(xs, axis=0)
    x_all = _ln_act(x_all, ln0g_ref[...], ln0b_ref[...], _elu)

    h_all = jnp.dot(x_all, W1w_ref[...].T, preferred_element_type=f32) \
        + W1b_ref[...]
    s_all = lax.dot_general(h_all, a1s_ref[...], (((1,), (1,)), ((), ())),
                            preferred_element_type=f32)
    ones = jnp.ones((NP, 1), dtype=f32)
    hps = []
    rss = []
    for k in range(B):
        hk = h_all[k * NP:(k + 1) * NP, :]
        sk = s_all[k * NP:(k + 1) * NP, :]
        tk = lax.dot_general(a1t_ref[...], hk, (((1,), (1,)), ((), ())),
                             preferred_element_type=f32)
        mrow = im_ref[k:k + 1, :]
        vm = jnp.exp(tk * c) * mrow
        vam = jnp.exp(tk * (ALPHA * c)) * mrow
        E2 = jnp.maximum(jnp.exp(sk * c) * vm,
                         jnp.exp(sk * (ALPHA * c)) * vam)
        hps.append(jnp.dot(E2, hk, preferred_element_type=f32))
        rss.append(jnp.dot(E2, ones, preferred_element_type=f32))
    HP2 = jnp.concatenate(hps, axis=0)
    RS2 = imstack_ref[...] * jnp.concatenate(rss, axis=0)
    zero = RS2 == 0.0
    coef = imstack_ref[...] / jnp.where(zero, 1.0, RS2)
    x_all = jnp.where(zero, h_all, HP2 * coef)
    x_all = _ln_act(x_all, ln1g_ref[...], ln1b_ref[...], _elu)

    h_all = jnp.dot(x_all, Wow_ref[...].T, preferred_element_type=f32) \
        + Wob_ref[...]
    t_all = lax.dot_general(h_all, aot_ref[...], (((1,), (1,)), ((), ())),
                            preferred_element_type=f32)
    rows = []
    for k in range(B):
        hk = h_all[k * NP:(k + 1) * NP, :]
        hlast = hk[N - 1:N, :]
        s_last = jnp.sum(hlast * aos_ref[...])
        tc = t_all[k * NP:(k + 1) * NP, :]
        om = omstack_ref[k * NP:(k + 1) * NP, :]
        wcol = jnp.maximum(jnp.exp(s_last * c) * jnp.exp(tc * c),
                           jnp.exp(s_last * (ALPHA * c))
                           * jnp.exp(tc * (ALPHA * c))) * om
        rs = jnp.sum(wcol)
        zero_r = rs == 0.0
        rs = jnp.where(zero_r, 1.0, rs)
        hp = lax.dot_general(wcol, hk, (((0,), (0,)), ((), ())),
                             preferred_element_type=f32)
        rows.append((hp + jnp.where(zero_r, hlast, 0.0)) / rs)
    z = jnp.concatenate(rows, axis=0)

    z = _ln_act(z, lnog_ref[...], lnob_ref[...], _relu)
    z = jnp.dot(z, linow_ref[...].T, preferred_element_type=f32) \
        + linob_ref[...]
    z = _relu(z)
    z = _relu(jnp.dot(z, out1w_ref[...].T, preferred_element_type=f32)
              + out1b_ref[...])
    logits_ref[...] = jnp.sum(z * out2w_ref[...], axis=1, keepdims=True) \
        + out2b_ref[0, 0]


@jax.jit
def kernel(data, embed, W0_w, W0_b, a0, ln0_g, ln0_b, W1_w, W1_b, a1,
           ln1_g, ln1_b, Wo_w, Wo_b, ao, lno_g, lno_b, lino_w, lino_b,
           out1_w, out1_b, out2_w, out2_b):
    f32 = jnp.float32
    obs = (data != 0).astype(f32)
    m = jnp.pad(obs, ((0, 0), (1, NP - F - 1)))
    any_obs = jnp.any(data != 0, axis=1, keepdims=True)
    e0 = (jnp.arange(NP) == 0).astype(f32)[None, :]
    im = jnp.where(any_obs, m, e0)
    om = m.at[:, N - 1].set(1.0)

    imstack = im.reshape(B * NP, 1)
    omstack = om.reshape(B * NP, 1)
    imt = im.T
    embed_p = jnp.pad(embed, ((0, NP - N), (0, 0)))

    row = lambda v: v.reshape(1, -1)
    operands = (
        im, imstack, imt, omstack, embed_p,
        W0_w, row(W0_b), a0[:, :D], a0[:, D:], row(ln0_g), row(ln0_b),
        W1_w, row(W1_b), a1[:, :D], a1[:, D:], row(ln1_g), row(ln1_b),
        Wo_w, row(Wo_b), ao[:, :D], ao[:, D:], row(lno_g), row(lno_b),
        lino_w, row(lino_b), out1_w, row(out1_b), out2_w, row(out2_b),
    )

    logits = pl.pallas_call(
        _body,
        grid=(1,),
        in_specs=[pl.BlockSpec(x.shape, lambda i, nd=x.ndim: (0,) * nd)
                  for x in operands],
        out_specs=pl.BlockSpec((B, 1), lambda i: (0, 0)),
        out_shape=jax.ShapeDtypeStruct((B, 1), f32),
    )(*operands)
    return (logits, jnp.asarray(0.0))


# speedup vs baseline: 3.3858x; 1.1127x over previous
import math

import jax
import jax.numpy as jnp
from jax import lax
from jax.experimental import pallas as pl

B = 16
F = 511
N = 513
NP = 520
D = 128
ALPHA = 0.2
DH = math.sqrt(float(D))
EPS = 1e-6


def _ln_act(x, g, b, act):
    m = jnp.mean(x, axis=-1, keepdims=True)
    xc = x - m
    var = jnp.sum(xc * xc, axis=-1, keepdims=True) / (D - 1)
    inv = lax.rsqrt(jnp.maximum(var, 1e-30))
    return act(g * (xc * inv) + b)


def _elu(x):
    return jnp.where(x > 0, x, jnp.exp(x) - 1.0)


def _relu(x):
    return jnp.maximum(x, 0.0)


def _body(im_ref, imstack_ref, imt_ref, omstack_ref, embed_ref,
          W0w_ref, W0b_ref, a0s_ref, a0t_ref, ln0g_ref, ln0b_ref,
          W1w_ref, W1b_ref, a1s_ref, a1t_ref, ln1g_ref, ln1b_ref,
          Wow_ref, Wob_ref, aos_ref, aot_ref, lnog_ref, lnob_ref,
          linow_ref, linob_ref, out1w_ref, out1b_ref, out2w_ref, out2b_ref,
          logits_ref):
    f32 = jnp.float32
    c = 1.0 / DH

    x0 = embed_ref[...]
    h1 = jnp.dot(x0, W0w_ref[...].T, preferred_element_type=f32) + W0b_ref[...]
    s1 = lax.dot_general(h1, a0s_ref[...], (((1,), (1,)), ((), ())),
                         preferred_element_type=f32)
    t1 = lax.dot_general(a0t_ref[...], h1, (((1,), (1,)), ((), ())),
                         preferred_element_type=f32)
    E1 = jnp.maximum(jnp.exp(s1 * ((1.0 - ALPHA) * c)) * jnp.exp(t1 * c),
                     jnp.exp(t1 * (ALPHA * c)))

    HM = jnp.concatenate(
        [imstack_ref[k * NP:(k + 1) * NP, :] * h1 for k in range(B)], axis=1)
    HP = jnp.dot(E1, HM, preferred_element_type=f32)
    RS = jnp.dot(E1, imt_ref[...], preferred_element_type=f32)
    RSM = imt_ref[...] * RS
    ZERO = RSM == 0.0
    COEF = imt_ref[...] / jnp.where(ZERO, 1.0, RSM)
    ZF = ZERO.astype(f32)
    xs = []
    for k in range(B):
        xs.append(HP[:, k * D:(k + 1) * D] * COEF[:, k:k + 1]
                  + ZF[:, k:k + 1] * h1)
    x_all = jnp.concatenate(xs, axis=0)
    x_all = _ln_act(x_all, ln0g_ref[...], ln0b_ref[...], _elu)

    h_all = jnp.dot(x_all, W1w_ref[...].T, preferred_element_type=f32) \
        + W1b_ref[...]
    s_all = lax.dot_general(h_all, a1s_ref[...], (((1,), (1,)), ((), ())),
                            preferred_element_type=f32)
    h_aug = jnp.concatenate([h_all, jnp.ones((B * NP, 8), dtype=f32)], axis=1)
    hps = []
    rss = []
    for k in range(B):
        hk = h_all[k * NP:(k + 1) * NP, :]
        sk = s_all[k * NP:(k + 1) * NP, :]
        tk = lax.dot_general(a1t_ref[...], hk, (((1,), (1,)), ((), ())),
                             preferred_element_type=f32)
        mrow = im_ref[k:k + 1, :]
        vm = jnp.exp(tk * c) * mrow
        vam = jnp.exp(tk * (ALPHA * c)) * mrow
        E2 = jnp.maximum(jnp.exp(sk * ((1.0 - ALPHA) * c)) * vm, vam)
        agg = jnp.dot(E2, h_aug[k * NP:(k + 1) * NP, :],
                      preferred_element_type=f32)
        hps.append(agg[:, :D])
        rss.append(agg[:, D:D + 1])
    HP2 = jnp.concatenate(hps, axis=0)
    RSS = jnp.concatenate(rss, axis=1)
    RSM2 = imt_ref[...] * RSS
    ZERO2 = RSM2 == 0.0
    COEF2 = imt_ref[...] / jnp.where(ZERO2, 1.0, RSM2)
    ZF2 = ZERO2.astype(f32)
    xs = []
    for k in range(B):
        xs.append(HP2[k * NP:(k + 1) * NP, :] * COEF2[:, k:k + 1]
                  + ZF2[:, k:k + 1] * h_all[k * NP:(k + 1) * NP, :])
    x_all = jnp.concatenate(xs, axis=0)
    x_all = _ln_act(x_all, ln1g_ref[...], ln1b_ref[...], _elu)

    h_all = jnp.dot(x_all, Wow_ref[...].T, preferred_element_type=f32) \
        + Wob_ref[...]
    t_all = lax.dot_general(h_all, aot_ref[...], (((1,), (1,)), ((), ())),
                            preferred_element_type=f32)
    rows = []
    for k in range(B):
        hk = h_all[k * NP:(k + 1) * NP, :]
        hlast = hk[N - 1:N, :]
        s_last = jnp.sum(hlast * aos_ref[...])
        tc = t_all[k * NP:(k + 1) * NP, :]
        om = omstack_ref[k * NP:(k + 1) * NP, :]
        wcol = jnp.maximum(jnp.exp(s_last * ((1.0 - ALPHA) * c))
                           * jnp.exp(tc * c),
                           jnp.exp(tc * (ALPHA * c))) * om
        rs = jnp.sum(wcol)
        zero_r = rs == 0.0
        rs = jnp.where(zero_r, 1.0, rs)
        hp = lax.dot_general(wcol, hk, (((0,), (0,)), ((), ())),
                             preferred_element_type=f32)
        rows.append((hp + jnp.where(zero_r, hlast, 0.0)) / rs)
    z = jnp.concatenate(rows, axis=0)

    z = _ln_act(z, lnog_ref[...], lnob_ref[...], _relu)
    z = jnp.dot(z, linow_ref[...].T, preferred_element_type=f32) \
        + linob_ref[...]
    z = _relu(z)
    z = _relu(jnp.dot(z, out1w_ref[...].T, preferred_element_type=f32)
              + out1b_ref[...])
    logits_ref[...] = jnp.sum(z * out2w_ref[...], axis=1, keepdims=True) \
        + out2b_ref[0, 0]


@jax.jit
def kernel(data, embed, W0_w, W0_b, a0, ln0_g, ln0_b, W1_w, W1_b, a1,
           ln1_g, ln1_b, Wo_w, Wo_b, ao, lno_g, lno_b, lino_w, lino_b,
           out1_w, out1_b, out2_w, out2_b):
    f32 = jnp.float32
    obs = (data != 0).astype(f32)
    m = jnp.pad(obs, ((0, 0), (1, NP - F - 1)))
    any_obs = jnp.any(data != 0, axis=1, keepdims=True)
    e0 = (jnp.arange(NP) == 0).astype(f32)[None, :]
    im = jnp.where(any_obs, m, e0)
    om = m.at[:, N - 1].set(1.0)

    imstack = im.reshape(B * NP, 1)
    omstack = om.reshape(B * NP, 1)
    imt = im.T
    embed_p = jnp.pad(embed, ((0, NP - N), (0, 0)))

    row = lambda v: v.reshape(1, -1)
    operands = (
        im, imstack, imt, omstack, embed_p,
        W0_w, row(W0_b), a0[:, :D], a0[:, D:], row(ln0_g), row(ln0_b),
        W1_w, row(W1_b), a1[:, :D], a1[:, D:], row(ln1_g), row(ln1_b),
        Wo_w, row(Wo_b), ao[:, :D], ao[:, D:], row(lno_g), row(lno_b),
        lino_w, row(lino_b), out1_w, row(out1_b), out2_w, row(out2_b),
    )

    logits = pl.pallas_call(
        _body,
        grid=(1,),
        in_specs=[pl.BlockSpec(x.shape, lambda i, nd=x.ndim: (0,) * nd)
                  for x in operands],
        out_specs=pl.BlockSpec((B, 1), lambda i: (0, 0)),
        out_shape=jax.ShapeDtypeStruct((B, 1), f32),
    )(*operands)
    return (logits, jnp.asarray(0.0))
